# Initial kernel scaffold; baseline (speedup 1.0000x reference)
import jax
import jax.numpy as jnp
from jax.experimental import pallas as pl


def kernel(x, edge_index, W_self, b_self, W_neigh, b_neigh):
    raise NotImplementedError("write your pallas kernel here")


# speedup vs baseline: 7.5565x; 7.5565x over previous
import functools

import jax
import jax.numpy as jnp
from jax import lax
from jax.experimental import pallas as pl
from jax.experimental.pallas import tpu as pltpu
from jax.experimental.pallas import tpu_sc as plsc

_N = 10000
_E = 320000
_D = 128
_DP = 144
_NC = 2
_NS = 16
_NW = _NC * _NS
_EPW = _E // _NW
_B = 125
_NCH = _EPW // _B
_NP = 10240
_RPT = _NP // _NS

_mesh = plsc.VectorSubcoreMesh(core_axis_name="c", subcore_axis_name="s")


@functools.partial(
    pl.kernel,
    out_type=jax.ShapeDtypeStruct((_NC, _NP, _DP), jnp.float32),
    mesh=_mesh,
    scratch_types=[
        pltpu.VMEM_SHARED((_NP, _DP), jnp.float32),
        pltpu.VMEM((_NCH, _B), jnp.int32),
        pltpu.VMEM((_NCH, _B), jnp.int32),
        pltpu.VMEM((_B, _DP), jnp.float32),
        pltpu.SemaphoreType.DMA,
    ],
    compiler_params=pltpu.CompilerParams(use_tc_tiling_on_sc=False),
)
def _sc_aggregate(xa, src, dst, zeros, out, acc, src_v, dst_v, rows, sem):
    c = lax.axis_index("c")
    s = lax.axis_index("s")
    wid = s * _NC + c
    off = pl.multiple_of(s * _RPT, 8)
    pltpu.sync_copy(zeros.at[pl.ds(off, _RPT)], acc.at[pl.ds(off, _RPT)])
    pltpu.sync_copy(src.at[wid], src_v)
    pltpu.sync_copy(dst.at[wid], dst_v)
    plsc.subcore_barrier()

    def body(j, carry):
        pltpu.async_copy(xa.at[src_v.at[j]], rows, sem).wait()
        pltpu.sync_copy(rows, acc.at[dst_v.at[j]], add=True)
        return carry

    lax.fori_loop(0, _NCH, body, 0)
    plsc.subcore_barrier()
    pltpu.sync_copy(acc.at[pl.ds(off, _RPT)], out.at[c, pl.ds(off, _RPT)])


_RB = 1000


def _tc_body(x_ref, p0_ref, p1_ref, ws_ref, wn_ref, bias_ref, o_ref):
    p = p0_ref[...] + p1_ref[...]
    deg = jnp.sum(p[:, _D:], axis=1, keepdims=True)
    h = p[:, :_D] * (1.0 / jnp.maximum(deg, 1.0))
    o_ref[...] = (
        jnp.dot(x_ref[...], ws_ref[...], preferred_element_type=jnp.float32)
        + jnp.dot(h, wn_ref[...], preferred_element_type=jnp.float32)
        + bias_ref[...]
    )


_tc_dense = pl.pallas_call(
    _tc_body,
    grid=(_N // _RB,),
    in_specs=[
        pl.BlockSpec((_RB, _D), lambda i: (i, 0)),
        pl.BlockSpec((_RB, _DP), lambda i: (i, 0)),
        pl.BlockSpec((_RB, _DP), lambda i: (i, 0)),
        pl.BlockSpec((_D, _D), lambda i: (0, 0)),
        pl.BlockSpec((_D, _D), lambda i: (0, 0)),
        pl.BlockSpec((1, _D), lambda i: (0, 0)),
    ],
    out_specs=pl.BlockSpec((_RB, _D), lambda i: (i, 0)),
    out_shape=jax.ShapeDtypeStruct((_N, _D), jnp.float32),
)


def kernel(x, edge_index, W_self, b_self, W_neigh, b_neigh):
    ei = edge_index.astype(jnp.int32)
    src = ei[0].reshape(_NW, _NCH, _B)
    dst = ei[1].reshape(_NW, _NCH, _B)
    xa = jnp.concatenate(
        [x, jnp.ones((_N, 1), jnp.float32), jnp.zeros((_N, _DP - _D - 1), jnp.float32)],
        axis=1,
    )
    zeros = jnp.zeros((_NP, _DP), jnp.float32)
    partials = _sc_aggregate(xa, src, dst, zeros)
    bias = (b_self + b_neigh)[None, :]
    return _tc_dense(x, partials[0], partials[1], W_self.T, W_neigh.T, bias)
